# baseline (speedup 1.0000x reference)
import jax
import jax.numpy as jnp
from jax import lax
from jax.experimental import pallas as pl
from jax.experimental.pallas import tpu as pltpu
from jax.experimental.pallas import tpu_sc as plsc

B, S, D = 4, 4096, 2048
N = B * S
K = 1024
HL = 1024
HH = 8192
NW = 32
RPW = (B * K) // NW

_T_BLK = 512
_HT_BLK = 512
_HH_BLK = 512
N_TAB = N + B * K


def _gelu(h):
    return 0.5 * h * (1.0 + lax.erf(h * (2.0 ** -0.5)))


def _light_body(x_ref, g_ref, w1_ref, b1_ref, w2_ref, b2_ref,
                wh1_ref, wh2_ref, light_ref, wh1o_ref, wh2o_ref):
    wh1o_ref[...] = wh1_ref[...].astype(jnp.bfloat16)
    wh2o_ref[...] = wh2_ref[...].astype(jnp.bfloat16)
    xb = x_ref[...]
    nrm = jnp.sqrt(jnp.sum(xb * xb, axis=1, keepdims=True))
    normed = xb / jnp.maximum(nrm, 1e-12) * (D ** 0.5) * g_ref[...]
    h = jnp.dot(normed, w1_ref[...], preferred_element_type=jnp.float32)
    h = _gelu(h + b1_ref[...])
    light_ref[...] = (
        jnp.dot(h, w2_ref[...], preferred_element_type=jnp.float32)
        + b2_ref[...])


def _light_ffn(x2d, gamma_l, w1, b1, w2, b2, wh1, wh2):
    nblk = N // _T_BLK
    c1 = HH // nblk
    r2 = HH // nblk
    return pl.pallas_call(
        _light_body,
        grid=(nblk,),
        in_specs=[
            pl.BlockSpec((_T_BLK, D), lambda t: (t, 0)),
            pl.BlockSpec((1, D), lambda t: (0, 0)),
            pl.BlockSpec((D, HL), lambda t: (0, 0)),
            pl.BlockSpec((1, HL), lambda t: (0, 0)),
            pl.BlockSpec((HL, D), lambda t: (0, 0)),
            pl.BlockSpec((1, D), lambda t: (0, 0)),
            pl.BlockSpec((D, c1), lambda t: (0, t)),
            pl.BlockSpec((r2, D), lambda t: (t, 0)),
        ],
        out_specs=[
            pl.BlockSpec((_T_BLK, D), lambda t: (t, 0)),
            pl.BlockSpec((D, c1), lambda t: (0, t)),
            pl.BlockSpec((r2, D), lambda t: (t, 0)),
        ],
        out_shape=[
            jax.ShapeDtypeStruct((N, D), jnp.float32),
            jax.ShapeDtypeStruct((D, HH), jnp.bfloat16),
            jax.ShapeDtypeStruct((HH, D), jnp.bfloat16),
        ],
        compiler_params=pltpu.CompilerParams(
            dimension_semantics=("arbitrary",)),
    )(x2d, gamma_l.reshape(1, D), w1, b1.reshape(1, HL),
      w2, b2.reshape(1, D), wh1, wh2)


def _cumsum_lanes(m):
    k = 1
    while k < S:
        shifted = jnp.concatenate(
            [jnp.zeros((B, k), jnp.float32), m[:, :S - k]], axis=1)
        m = m + shifted
        k *= 2
    return m


def _route_body(s_ref, idx_ref):
    scores = s_ref[...]
    bits = jax.lax.bitcast_convert_type(scores, jnp.int32)
    keys = jnp.where(bits < 0, bits ^ jnp.int32(0x7FFFFFFF), bits)

    lo = jnp.min(keys, axis=1, keepdims=True)
    hi = jnp.max(keys, axis=1, keepdims=True) + 1

    def bisect(_, carry):
        lo, hi = carry
        mid = (lo & hi) + ((lo ^ hi) >> 1)
        cnt = jnp.sum((keys >= mid).astype(jnp.int32), axis=1, keepdims=True)
        ok = cnt >= K
        return jnp.where(ok, mid, lo), jnp.where(ok, hi, mid)

    lo, hi = lax.fori_loop(0, 32, bisect, (lo, hi))
    thr = lo

    gt = keys > thr
    tie = keys == thr
    n_gt = jnp.sum(gt.astype(jnp.int32), axis=1, keepdims=True)
    extra = (K - n_gt).astype(jnp.float32)
    tie_f = tie.astype(jnp.float32)
    c_tie = _cumsum_lanes(tie_f)
    total_tie = jnp.sum(tie_f, axis=1, keepdims=True)
    ties_after = total_tie - c_tie
    sel = gt | (tie & (ties_after < extra))

    h = _cumsum_lanes(sel.astype(jnp.float32))

    iota8 = lax.broadcasted_iota(jnp.int32, (8, 1), 0).astype(jnp.float32)
    for b in range(B):
        hb = h[b:b + 1, :]

        def jc_body(jc, _, hb=hb, b=b):
            jv = jc.astype(jnp.float32) * 8.0 + iota8
            cmp = (hb <= jv).astype(jnp.float32)
            pos = jnp.sum(cmp, axis=1, keepdims=True)
            idx_ref[pl.ds(jc * 8, 8), b:b + 1] = (
                pos.astype(jnp.int32) + b * S)
            return 0

        lax.fori_loop(0, K // 8, jc_body, 0)


def _route(scores_bs):
    return pl.pallas_call(
        _route_body,
        out_shape=jax.ShapeDtypeStruct((K, B), jnp.int32),
    )(scores_bs)


_CH = 32


def _sc_gather(x2d, gidx):
    cpw = RPW // _CH

    def body(x_hbm, gidx_hbm, xsel_hbm, idx_v, rows_v, sem):
        wid = lax.axis_index("s") * 2 + lax.axis_index("c")
        base = wid * RPW
        for c in range(cpw):
            off = base + c * _CH
            pltpu.sync_copy(gidx_hbm.at[pl.ds(off, _CH)], idx_v)
            pltpu.async_copy(x_hbm.at[idx_v], rows_v, sem).wait()
            pltpu.sync_copy(rows_v, xsel_hbm.at[pl.ds(off, _CH)])

    f = pl.kernel(
        body,
        out_type=jax.ShapeDtypeStruct((B * K, D), jnp.float32),
        mesh=plsc.VectorSubcoreMesh(core_axis_name="c", subcore_axis_name="s"),
        scratch_types=[
            pltpu.VMEM((_CH,), jnp.int32),
            pltpu.VMEM((_CH, D), jnp.float32),
            pltpu.SemaphoreType.DMA,
        ],
    )
    return f(x2d, gidx)


def _heavy_body(x_ref, g_ref, w1_ref, b1_ref, w2_ref, b2_ref,
                out_ref, normx_ref, acc_ref):
    hi = pl.program_id(1)
    nh = pl.num_programs(1)

    @pl.when(hi == 0)
    def _():
        xb = x_ref[...]
        nrm = jnp.sqrt(jnp.sum(xb * xb, axis=1, keepdims=True))
        normx_ref[...] = (xb / jnp.maximum(nrm, 1e-12) * (D ** 0.5)
                          * g_ref[...]).astype(jnp.bfloat16)
        acc_ref[...] = jnp.zeros_like(acc_ref)

    h = jnp.dot(normx_ref[...], w1_ref[...],
                preferred_element_type=jnp.float32)
    h = _gelu(h + b1_ref[...])
    acc_ref[...] += jnp.dot(h.astype(jnp.bfloat16), w2_ref[...],
                            preferred_element_type=jnp.float32)

    @pl.when(hi == nh - 1)
    def _():
        out_ref[...] = acc_ref[...] + b2_ref[...]


def _heavy_ffn(x_sel, gamma_h, w1_bf, b1, w2_bf, b2):
    nt = (B * K) // _HT_BLK
    nh = HH // _HH_BLK
    return pl.pallas_call(
        _heavy_body,
        grid=(nt, nh),
        in_specs=[
            pl.BlockSpec((_HT_BLK, D), lambda t, h: (t, 0)),
            pl.BlockSpec((1, D), lambda t, h: (0, 0)),
            pl.BlockSpec((D, _HH_BLK), lambda t, h: (0, h)),
            pl.BlockSpec((1, _HH_BLK), lambda t, h: (0, h)),
            pl.BlockSpec((_HH_BLK, D), lambda t, h: (h, 0)),
            pl.BlockSpec((1, D), lambda t, h: (0, 0)),
        ],
        out_specs=pl.BlockSpec((_HT_BLK, D), lambda t, h: (t, 0)),
        out_shape=jax.ShapeDtypeStruct((B * K, D), jnp.float32),
        scratch_shapes=[
            pltpu.VMEM((_HT_BLK, D), jnp.bfloat16),
            pltpu.VMEM((_HT_BLK, D), jnp.float32),
        ],
        compiler_params=pltpu.CompilerParams(
            dimension_semantics=("arbitrary", "arbitrary")),
    )(x_sel, gamma_h.reshape(1, D), w1_bf, b1.reshape(1, HH), w2_bf,
      b2.reshape(1, D))


def _add_body(a_ref, b_ref, o_ref):
    o_ref[...] = a_ref[...] + b_ref[...]


def _add_rows(a, b):
    nblk = (B * K) // _T_BLK
    return pl.pallas_call(
        _add_body,
        grid=(nblk,),
        in_specs=[
            pl.BlockSpec((_T_BLK, D), lambda t: (t, 0)),
            pl.BlockSpec((_T_BLK, D), lambda t: (t, 0)),
        ],
        out_specs=pl.BlockSpec((_T_BLK, D), lambda t: (t, 0)),
        out_shape=jax.ShapeDtypeStruct((B * K, D), jnp.float32),
        compiler_params=pltpu.CompilerParams(
            dimension_semantics=("arbitrary",)),
    )(a, b)


def _sc_scatter(dst_ref, rows, gidx):
    def body(d_ref, rows_hbm, gidx_hbm, idx_v, rows_v, sem):
        wid = lax.axis_index("s") * 2 + lax.axis_index("c")
        base = wid * RPW
        for c in range(RPW // _CH):
            off = base + c * _CH
            pltpu.sync_copy(gidx_hbm.at[pl.ds(off, _CH)], idx_v)
            pltpu.sync_copy(rows_hbm.at[pl.ds(off, _CH)], rows_v)
            pltpu.async_copy(rows_v, d_ref.at[idx_v], sem).wait()

    f = pl.kernel(
        body,
        out_type=(),
        mesh=plsc.VectorSubcoreMesh(core_axis_name="c", subcore_axis_name="s"),
        scratch_types=[
            pltpu.VMEM((_CH,), jnp.int32),
            pltpu.VMEM((_CH, D), jnp.float32),
            pltpu.SemaphoreType.DMA,
        ],
    )
    f(dst_ref, rows, gidx)


def kernel(x, routing_token, gamma_l, W1_l, b1_l, W2_l, b2_l,
           gamma_h, W1_h, b1_h, W2_h, b2_h):
    x2d = x.reshape(N, D)
    scores = jnp.einsum('bnd,rd->brn', x, routing_token).reshape(B, S)
    idx_t = _route(scores)
    gidx = idx_t.T.reshape(-1)
    x_sel = _sc_gather(x2d, gidx)
    light2d, w1h_bf, w2h_bf = _light_ffn(x2d, gamma_l, W1_l, b1_l, W2_l,
                                         b2_l, W1_h, W2_h)
    light_sel = _sc_gather(light2d, gidx)
    rows = _heavy_ffn(x_sel, gamma_h, w1h_bf, b1_h, w2h_bf, b2_h)
    rows = _add_rows(rows, light_sel)
    out_ref = jax.new_ref(light2d)
    _sc_scatter(out_ref, rows, gidx)
    return out_ref[...].reshape(B, S, D)


# speedup vs baseline: 1.0220x; 1.0220x over previous
import jax
import jax.numpy as jnp
from jax import lax
from jax.experimental import pallas as pl
from jax.experimental.pallas import tpu as pltpu
from jax.experimental.pallas import tpu_sc as plsc

B, S, D = 4, 4096, 2048
N = B * S
K = 1024
HL = 1024
HH = 8192
NW = 32
RPW = (B * K) // NW

_T_BLK = 512
_HT_BLK = 512
_HH_BLK = 1024
N_TAB = N + B * K


def _gelu(h):
    return 0.5 * h * (1.0 + lax.erf(h * (2.0 ** -0.5)))


def _light_body(x_ref, g_ref, w1_ref, b1_ref, w2_ref, b2_ref,
                wh1_ref, wh2_ref, light_ref, wh1o_ref, wh2o_ref):
    wh1o_ref[...] = wh1_ref[...].astype(jnp.bfloat16)
    wh2o_ref[...] = wh2_ref[...].astype(jnp.bfloat16)
    xb = x_ref[...]
    nrm = jnp.sqrt(jnp.sum(xb * xb, axis=1, keepdims=True))
    normed = xb / jnp.maximum(nrm, 1e-12) * (D ** 0.5) * g_ref[...]
    h = jnp.dot(normed, w1_ref[...], preferred_element_type=jnp.float32)
    h = _gelu(h + b1_ref[...])
    light_ref[...] = (
        jnp.dot(h, w2_ref[...], preferred_element_type=jnp.float32)
        + b2_ref[...])


def _light_ffn(x2d, gamma_l, w1, b1, w2, b2, wh1, wh2):
    nblk = N // _T_BLK
    c1 = HH // nblk
    r2 = HH // nblk
    return pl.pallas_call(
        _light_body,
        grid=(nblk,),
        in_specs=[
            pl.BlockSpec((_T_BLK, D), lambda t: (t, 0)),
            pl.BlockSpec((1, D), lambda t: (0, 0)),
            pl.BlockSpec((D, HL), lambda t: (0, 0)),
            pl.BlockSpec((1, HL), lambda t: (0, 0)),
            pl.BlockSpec((HL, D), lambda t: (0, 0)),
            pl.BlockSpec((1, D), lambda t: (0, 0)),
            pl.BlockSpec((D, c1), lambda t: (0, t)),
            pl.BlockSpec((r2, D), lambda t: (t, 0)),
        ],
        out_specs=[
            pl.BlockSpec((_T_BLK, D), lambda t: (t, 0)),
            pl.BlockSpec((D, c1), lambda t: (0, t)),
            pl.BlockSpec((r2, D), lambda t: (t, 0)),
        ],
        out_shape=[
            jax.ShapeDtypeStruct((N, D), jnp.float32),
            jax.ShapeDtypeStruct((D, HH), jnp.bfloat16),
            jax.ShapeDtypeStruct((HH, D), jnp.bfloat16),
        ],
        compiler_params=pltpu.CompilerParams(
            dimension_semantics=("arbitrary",)),
    )(x2d, gamma_l.reshape(1, D), w1, b1.reshape(1, HL),
      w2, b2.reshape(1, D), wh1, wh2)


def _cumsum_lanes(m):
    k = 1
    while k < S:
        shifted = jnp.concatenate(
            [jnp.zeros((B, k), jnp.float32), m[:, :S - k]], axis=1)
        m = m + shifted
        k *= 2
    return m


def _route_body(s_ref, idx_ref):
    scores = s_ref[...]
    bits = jax.lax.bitcast_convert_type(scores, jnp.int32)
    keys = jnp.where(bits < 0, bits ^ jnp.int32(0x7FFFFFFF), bits)

    lo = jnp.min(keys, axis=1, keepdims=True)
    hi = jnp.max(keys, axis=1, keepdims=True) + 1

    def bisect(_, carry):
        lo, hi = carry
        mid = (lo & hi) + ((lo ^ hi) >> 1)
        cnt = jnp.sum((keys >= mid).astype(jnp.int32), axis=1, keepdims=True)
        ok = cnt >= K
        return jnp.where(ok, mid, lo), jnp.where(ok, hi, mid)

    lo, hi = lax.fori_loop(0, 32, bisect, (lo, hi))
    thr = lo

    gt = keys > thr
    tie = keys == thr
    n_gt = jnp.sum(gt.astype(jnp.int32), axis=1, keepdims=True)
    extra = (K - n_gt).astype(jnp.float32)
    tie_f = tie.astype(jnp.float32)
    c_tie = _cumsum_lanes(tie_f)
    total_tie = jnp.sum(tie_f, axis=1, keepdims=True)
    ties_after = total_tie - c_tie
    sel = gt | (tie & (ties_after < extra))

    h = _cumsum_lanes(sel.astype(jnp.float32))

    iota8 = lax.broadcasted_iota(jnp.int32, (8, 1), 0).astype(jnp.float32)
    for b in range(B):
        hb = h[b:b + 1, :]

        def jc_body(jc, _, hb=hb, b=b):
            jv = jc.astype(jnp.float32) * 8.0 + iota8
            cmp = (hb <= jv).astype(jnp.float32)
            pos = jnp.sum(cmp, axis=1, keepdims=True)
            idx_ref[pl.ds(jc * 8, 8), b:b + 1] = (
                pos.astype(jnp.int32) + b * S)
            return 0

        lax.fori_loop(0, K // 8, jc_body, 0)


def _route(scores_bs):
    return pl.pallas_call(
        _route_body,
        out_shape=jax.ShapeDtypeStruct((K, B), jnp.int32),
    )(scores_bs)


_CH = 16


def _sc_gather(x2d, gidx):
    cpw = RPW // _CH

    def body(x_hbm, gidx_hbm, xsel_hbm, idx_v, rows_v, sem):
        wid = lax.axis_index("s") * 2 + lax.axis_index("c")
        base = wid * RPW
        for c in range(cpw):
            off = base + c * _CH
            pltpu.sync_copy(gidx_hbm.at[pl.ds(off, _CH)], idx_v)
            pltpu.async_copy(x_hbm.at[idx_v], rows_v, sem).wait()
            pltpu.sync_copy(rows_v, xsel_hbm.at[pl.ds(off, _CH)])

    f = pl.kernel(
        body,
        out_type=jax.ShapeDtypeStruct((B * K, D), jnp.float32),
        mesh=plsc.VectorSubcoreMesh(core_axis_name="c", subcore_axis_name="s"),
        scratch_types=[
            pltpu.VMEM((_CH,), jnp.int32),
            pltpu.VMEM((_CH, D), jnp.float32),
            pltpu.SemaphoreType.DMA,
        ],
    )
    return f(x2d, gidx)


def _heavy_body(x_ref, g_ref, w1_ref, b1_ref, w2_ref, b2_ref,
                out_ref, normx_ref, acc_ref):
    hi = pl.program_id(1)
    nh = pl.num_programs(1)

    @pl.when(hi == 0)
    def _():
        xb = x_ref[...]
        nrm = jnp.sqrt(jnp.sum(xb * xb, axis=1, keepdims=True))
        normx_ref[...] = (xb / jnp.maximum(nrm, 1e-12) * (D ** 0.5)
                          * g_ref[...]).astype(jnp.bfloat16)
        acc_ref[...] = jnp.zeros_like(acc_ref)

    h = jnp.dot(normx_ref[...], w1_ref[...],
                preferred_element_type=jnp.float32)
    h = _gelu(h + b1_ref[...])
    acc_ref[...] += jnp.dot(h.astype(jnp.bfloat16), w2_ref[...],
                            preferred_element_type=jnp.float32)

    @pl.when(hi == nh - 1)
    def _():
        out_ref[...] = acc_ref[...] + b2_ref[...]


def _heavy_ffn(x_sel, gamma_h, w1_bf, b1, w2_bf, b2):
    nt = (B * K) // _HT_BLK
    nh = HH // _HH_BLK
    return pl.pallas_call(
        _heavy_body,
        grid=(nt, nh),
        in_specs=[
            pl.BlockSpec((_HT_BLK, D), lambda t, h: (t, 0)),
            pl.BlockSpec((1, D), lambda t, h: (0, 0)),
            pl.BlockSpec((D, _HH_BLK), lambda t, h: (0, h)),
            pl.BlockSpec((1, _HH_BLK), lambda t, h: (0, h)),
            pl.BlockSpec((_HH_BLK, D), lambda t, h: (h, 0)),
            pl.BlockSpec((1, D), lambda t, h: (0, 0)),
        ],
        out_specs=pl.BlockSpec((_HT_BLK, D), lambda t, h: (t, 0)),
        out_shape=jax.ShapeDtypeStruct((B * K, D), jnp.float32),
        scratch_shapes=[
            pltpu.VMEM((_HT_BLK, D), jnp.bfloat16),
            pltpu.VMEM((_HT_BLK, D), jnp.float32),
        ],
        compiler_params=pltpu.CompilerParams(
            dimension_semantics=("arbitrary", "arbitrary")),
    )(x_sel, gamma_h.reshape(1, D), w1_bf, b1.reshape(1, HH), w2_bf,
      b2.reshape(1, D))


def _add_body(a_ref, b_ref, o_ref):
    o_ref[...] = a_ref[...] + b_ref[...]


def _add_rows(a, b):
    nblk = (B * K) // _T_BLK
    return pl.pallas_call(
        _add_body,
        grid=(nblk,),
        in_specs=[
            pl.BlockSpec((_T_BLK, D), lambda t: (t, 0)),
            pl.BlockSpec((_T_BLK, D), lambda t: (t, 0)),
        ],
        out_specs=pl.BlockSpec((_T_BLK, D), lambda t: (t, 0)),
        out_shape=jax.ShapeDtypeStruct((B * K, D), jnp.float32),
        compiler_params=pltpu.CompilerParams(
            dimension_semantics=("arbitrary",)),
    )(a, b)


def _sc_scatter(dst_ref, rows, gidx):
    def body(d_ref, rows_hbm, gidx_hbm, idx_v, rows_v, sem):
        wid = lax.axis_index("s") * 2 + lax.axis_index("c")
        base = wid * RPW
        for c in range(RPW // _CH):
            off = base + c * _CH
            pltpu.sync_copy(gidx_hbm.at[pl.ds(off, _CH)], idx_v)
            pltpu.sync_copy(rows_hbm.at[pl.ds(off, _CH)], rows_v)
            pltpu.async_copy(rows_v, d_ref.at[idx_v], sem).wait()

    f = pl.kernel(
        body,
        out_type=(),
        mesh=plsc.VectorSubcoreMesh(core_axis_name="c", subcore_axis_name="s"),
        scratch_types=[
            pltpu.VMEM((_CH,), jnp.int32),
            pltpu.VMEM((_CH, D), jnp.float32),
            pltpu.SemaphoreType.DMA,
        ],
    )
    f(dst_ref, rows, gidx)


def kernel(x, routing_token, gamma_l, W1_l, b1_l, W2_l, b2_l,
           gamma_h, W1_h, b1_h, W2_h, b2_h):
    x2d = x.reshape(N, D)
    scores = jnp.einsum('bnd,rd->brn', x, routing_token).reshape(B, S)
    idx_t = _route(scores)
    gidx = idx_t.T.reshape(-1)
    x_sel = _sc_gather(x2d, gidx)
    light2d, w1h_bf, w2h_bf = _light_ffn(x2d, gamma_l, W1_l, b1_l, W2_l,
                                         b2_l, W1_h, W2_h)
    light_sel = _sc_gather(light2d, gidx)
    rows = _heavy_ffn(x_sel, gamma_h, w1h_bf, b1_h, w2h_bf, b2_h)
    rows = _add_rows(rows, light_sel)
    out_ref = jax.new_ref(light2d)
    _sc_scatter(out_ref, rows, gidx)
    return out_ref[...].reshape(B, S, D)
